# baseline (speedup 1.0000x reference)
import functools

import jax
import jax.numpy as jnp
from jax import lax
from jax.experimental import pallas as pl
from jax.experimental.pallas import tpu as pltpu
from jax.experimental.pallas import tpu_sc as plsc

NC = 2
NS = 16
LANES = 16
GRP = 128
CH = 1024
KG = CH // GRP
PADROWS = 64


def _chunk_loop(s, ept, nch, src_ref, agg_ref, row_hbm, col_hbm,
                ridx, cidx, msg, sem_g, sem_s):
  g0_tile = s * (ept // GRP)

  def chunk_body(t, carry):
    for b in range(2):
      g0 = g0_tile + (2 * t + b) * KG
      pltpu.sync_copy(col_hbm.at[pl.ds(g0, KG)], cidx.at[b])
      if src_ref is not None:
        pltpu.sync_copy(row_hbm.at[pl.ds(g0, KG)], ridx.at[b])
    if src_ref is not None:
      gathers = [
          pltpu.async_copy(src_ref.at[ridx.at[b, j]],
                           msg.at[b, pl.ds(j * GRP, GRP)], sem_g)
          for b in range(2) for j in range(KG)
      ]
    scatters = []
    for i, (b, j) in enumerate([(b, j) for b in range(2)
                                for j in range(KG)]):
      if src_ref is not None:
        gathers[i].wait()
      scatters.append(
          pltpu.async_copy(msg.at[b, pl.ds(j * GRP, GRP)],
                           agg_ref.at[cidx.at[b, j]], sem_s, add=True))
    for sc_ in scatters:
      sc_.wait()
    return carry
  lax.fori_loop(0, nch // 2, chunk_body, 0)


def _make_sc_deg(n_agg, e_pad):
  nw = NC * NS
  ept = e_pad // nw
  nch = ept // CH
  assert ept % (2 * CH) == 0 and n_agg % (NS * 8) == 0
  zpt = n_agg // NS
  mesh = plsc.VectorSubcoreMesh(core_axis_name="c", subcore_axis_name="s")

  @functools.partial(
      pl.kernel,
      out_type=jax.ShapeDtypeStruct((NC, n_agg, LANES), jnp.float32),
      mesh=mesh,
      compiler_params=pltpu.CompilerParams(use_tc_tiling_on_sc=False),
      scratch_types=[
          pltpu.VMEM((2, KG, GRP), jnp.int32),
          pltpu.VMEM((2, CH, LANES), jnp.float32),
          pltpu.VMEM((n_agg // NS, LANES), jnp.float32),
          pltpu.VMEM_SHARED((n_agg, LANES), jnp.float32),
          pltpu.SemaphoreType.DMA,
      ],
  )
  def sc_deg(col_hbm, out_hbm, cidx, msg, stage, agg, sem_s):
    c = lax.axis_index("c")
    s = lax.axis_index("s")
    wid = s * NC + c

    def zero_body(i, carry):
      stage[i, :] = jnp.zeros((LANES,), jnp.float32)
      return carry
    lax.fori_loop(0, zpt, zero_body, 0)
    pltpu.sync_copy(stage, agg.at[pl.ds(s * zpt, zpt)])

    def ones_body(i, carry):
      msg[0, i, :] = jnp.ones((LANES,), jnp.float32)
      msg[1, i, :] = jnp.ones((LANES,), jnp.float32)
      return carry
    lax.fori_loop(0, CH, ones_body, 0)
    plsc.subcore_barrier()

    _chunk_loop(wid, ept, nch, None, agg, None, col_hbm,
                None, cidx, msg, None, sem_s)

    plsc.subcore_barrier()
    pltpu.sync_copy(agg.at[pl.ds(s * zpt, zpt)], stage)
    pltpu.sync_copy(stage, out_hbm.at[c, pl.ds(s * zpt, zpt)])

  return sc_deg


def _make_sc_big(n_agg, e_pad):
  ept = e_pad // NS
  nch = ept // CH
  assert ept % (2 * CH) == 0 and n_agg % (NS * NC * 8) == 0
  zpt = n_agg // NS
  qpt = n_agg // (NS * NC)
  mesh = plsc.VectorSubcoreMesh(core_axis_name="c", subcore_axis_name="s")

  @functools.partial(
      pl.kernel,
      out_type=(jax.ShapeDtypeStruct((n_agg, LANES), jnp.float32),
                jax.ShapeDtypeStruct((NC, n_agg, LANES), jnp.float32)),
      mesh=mesh,
      compiler_params=pltpu.CompilerParams(use_tc_tiling_on_sc=False),
      scratch_types=[
          pltpu.VMEM((2, KG, GRP), jnp.int32),
          pltpu.VMEM((2, KG, GRP), jnp.int32),
          pltpu.VMEM((2, CH, LANES), jnp.float32),
          pltpu.VMEM((n_agg // NS, LANES), jnp.float32),
          pltpu.VMEM((n_agg // NS, LANES), jnp.float32),
          pltpu.VMEM((n_agg // NS, LANES), jnp.float32),
          pltpu.VMEM((n_agg // NS, LANES), jnp.float32),
          pltpu.VMEM((n_agg // NS, LANES), jnp.float32),
          pltpu.VMEM_SHARED((n_agg, LANES), jnp.float32),
          pltpu.VMEM_SHARED((n_agg, LANES), jnp.float32),
          pltpu.SemaphoreType.DMA,
          pltpu.SemaphoreType.DMA,
      ],
  )
  def sc_big(zs0_hbm, h_hbm, dinv_hbm, ideg_hbm, row_hbm, col_hbm,
             z2_hbm, zs1_hbm,
             ridx, cidx, msg, hbuf, dbuf, ibuf, abuf, zbuf,
             agg1, agg2, sem_g, sem_s):
    c = lax.axis_index("c")
    s = lax.axis_index("s")

    def zero_body(i, carry):
      zbuf[i, :] = jnp.zeros((LANES,), jnp.float32)
      return carry
    lax.fori_loop(0, zpt, zero_body, 0)
    pltpu.sync_copy(zbuf, agg1.at[pl.ds(s * zpt, zpt)])
    pltpu.sync_copy(zbuf, agg2.at[pl.ds(s * zpt, zpt)])
    plsc.subcore_barrier()

    _chunk_loop(s, ept, nch, zs0_hbm, agg1, row_hbm, col_hbm,
                ridx, cidx, msg, sem_g, sem_s)
    plsc.subcore_barrier()

    r0 = s * zpt
    pltpu.sync_copy(h_hbm.at[pl.ds(r0, zpt)], hbuf)
    pltpu.sync_copy(dinv_hbm.at[pl.ds(r0, zpt)], dbuf)
    pltpu.sync_copy(ideg_hbm.at[pl.ds(r0, zpt)], ibuf)
    pltpu.sync_copy(agg1.at[pl.ds(r0, zpt)], abuf)

    def comb1(i, carry):
      hv = hbuf[i, :]
      dv = dbuf[i, :]
      z1 = 0.5 * (dv * abuf[i, :] + hv * ibuf[i, :]) + 0.5 * hv
      zbuf[i, :] = z1 * dv
      return carry
    lax.fori_loop(0, zpt, comb1, 0)
    pltpu.sync_copy(zbuf, zs1_hbm.at[c, pl.ds(r0, zpt)])
    plsc.subcore_barrier()

    _chunk_loop(s, ept, nch, zs1_hbm.at[c], agg2, row_hbm, col_hbm,
                ridx, cidx, msg, sem_g, sem_s)
    plsc.subcore_barrier()

    w0 = c * (n_agg // NC) + s * qpt
    pltpu.sync_copy(h_hbm.at[pl.ds(w0, qpt)], hbuf.at[pl.ds(0, qpt)])
    pltpu.sync_copy(dinv_hbm.at[pl.ds(w0, qpt)], dbuf.at[pl.ds(0, qpt)])
    pltpu.sync_copy(ideg_hbm.at[pl.ds(w0, qpt)], ibuf.at[pl.ds(0, qpt)])
    pltpu.sync_copy(agg1.at[pl.ds(w0, qpt)], abuf.at[pl.ds(0, qpt)])
    pltpu.sync_copy(agg2.at[pl.ds(w0, qpt)], msg.at[0, pl.ds(0, qpt)])

    def comb2(i, carry):
      hv = hbuf[i, :]
      dv = dbuf[i, :]
      iv = ibuf[i, :]
      z1 = 0.5 * (dv * abuf[i, :] + hv * iv) + 0.5 * hv
      zbuf[i, :] = 0.5 * (dv * msg[0, i, :] + z1 * iv) + 0.5 * hv
      return carry
    lax.fori_loop(0, qpt, comb2, 0)
    pltpu.sync_copy(zbuf.at[pl.ds(0, qpt)], z2_hbm.at[pl.ds(w0, qpt)])

  return sc_big


def _gelu(v):
  return 0.5 * v * (1.0 + lax.erf(v * (2.0 ** -0.5)))


def _ln(h, g, b):
  mu = jnp.mean(h, axis=-1, keepdims=True)
  d = h - mu
  var = jnp.mean(d * d, axis=-1, keepdims=True)
  return d * lax.rsqrt(var + 1e-5) * g + b


def _tc_pre_body(x_ref, w1_ref, b1_ref, g1_ref, bt1_ref, s0_ref,
                 zs_ref, h_ref, dinv_ref, ideg_ref):
  h = jnp.dot(x_ref[...], w1_ref[...], preferred_element_type=jnp.float32)
  h = _gelu(h + b1_ref[...])
  h = _ln(h, g1_ref[...], bt1_ref[...])
  deg = s0_ref[0] + s0_ref[1] + 1.0
  dinv = lax.rsqrt(deg)
  ideg = 1.0 / deg
  h_ref[...] = h
  zs_ref[...] = h * dinv
  dinv_ref[...] = dinv
  ideg_ref[...] = ideg


def _tc_post_body(z2_ref, g2_ref, bt2_ref, w2_ref, b2_ref, out_ref):
  t = _ln(_gelu(z2_ref[...]), g2_ref[...], bt2_ref[...])
  out_ref[...] = jnp.dot(t, w2_ref[...],
                         preferred_element_type=jnp.float32) + b2_ref[...]


def kernel(x, edge_index, W1, b1, g1, bt1, g2, bt2, W2, b2):
  n, din = x.shape
  hid = W1.shape[1]
  dout = W2.shape[1]
  assert hid == LANES
  e = edge_index.shape[1]

  span = NC * NS * 2 * CH
  e_pad = -(-e // span) * span
  pad = e_pad - e
  row = edge_index[0].astype(jnp.int32)
  col = edge_index[1].astype(jnp.int32)
  ar = jnp.arange(pad, dtype=jnp.int32)
  rowp = jnp.concatenate([row, ar % n]).reshape(e_pad // GRP, GRP)
  colp = jnp.concatenate([col, n + (ar % PADROWS)]).reshape(e_pad // GRP, GRP)
  n_agg = -(-(n + PADROWS) // 256) * 256

  s0 = _make_sc_deg(n_agg, e_pad)(colp)

  R = 2048
  grid = (n_agg // R,)
  row_spec = pl.BlockSpec((R, LANES), lambda i: (i, 0))
  part_spec = pl.BlockSpec((NC, R, LANES), lambda i: (0, i, 0))
  vec16 = pl.BlockSpec((1, LANES), lambda i: (0, 0))
  st = jax.ShapeDtypeStruct((n_agg, LANES), jnp.float32)

  zs0, h, dinv, ideg = pl.pallas_call(
      _tc_pre_body,
      grid=grid,
      in_specs=[
          pl.BlockSpec((R, din), lambda i: (i, 0)),
          pl.BlockSpec((din, LANES), lambda i: (0, 0)),
          vec16, vec16, vec16,
          part_spec,
      ],
      out_specs=[row_spec, row_spec, row_spec, row_spec],
      out_shape=[st, st, st, st],
  )(x, W1, b1.reshape(1, -1), g1.reshape(1, -1), bt1.reshape(1, -1), s0)

  z2, _ = _make_sc_big(n_agg, e_pad)(zs0, h, dinv, ideg, rowp, colp)

  out = pl.pallas_call(
      _tc_post_body,
      grid=grid,
      in_specs=[
          row_spec,
          vec16, vec16,
          pl.BlockSpec((LANES, dout), lambda i: (0, 0)),
          pl.BlockSpec((1, dout), lambda i: (0, 0)),
      ],
      out_specs=pl.BlockSpec((R, dout), lambda i: (i, 0)),
      out_shape=jax.ShapeDtypeStruct((n, dout), jnp.float32),
  )(z2, g2.reshape(1, -1), bt2.reshape(1, -1), W2, b2.reshape(1, -1))
  return out


# speedup vs baseline: 1.3690x; 1.3690x over previous
import functools

import jax
import jax.numpy as jnp
from jax import lax
from jax.experimental import pallas as pl
from jax.experimental.pallas import tpu as pltpu
from jax.experimental.pallas import tpu_sc as plsc

NC = 2
NS = 16
LANES = 16
GRP = 128
CH = 1024
KG = CH // GRP
PADROWS = 64


def _make_sc_round(n_agg, e_pad, with_gather):
  nw = NC * NS
  ept = e_pad // nw
  nch = ept // CH
  assert ept % (2 * CH) == 0 and n_agg % (NS * 8) == 0
  zpt = n_agg // NS
  mesh = plsc.VectorSubcoreMesh(core_axis_name="c", subcore_axis_name="s")

  ngrp = ept // GRP

  @functools.partial(
      pl.kernel,
      out_type=jax.ShapeDtypeStruct((NC, n_agg, LANES), jnp.float32),
      mesh=mesh,
      compiler_params=pltpu.CompilerParams(use_tc_tiling_on_sc=False),
      scratch_types=[
          pltpu.VMEM((ngrp, GRP), jnp.int32),
          pltpu.VMEM((ngrp, GRP), jnp.int32),
          pltpu.VMEM((2, CH, LANES), jnp.float32),
          pltpu.VMEM((n_agg // NS, LANES), jnp.float32),
          pltpu.VMEM_SHARED((n_agg, LANES), jnp.float32),
          pltpu.SemaphoreType.DMA,
          pltpu.SemaphoreType.DMA,
      ],
  )
  def sc_round(zs_hbm, row_hbm, col_hbm, out_hbm,
               ridx, cidx, msg, stage, agg, sem_g, sem_s):
    c = lax.axis_index("c")
    s = lax.axis_index("s")
    wid = s * NC + c

    g0_tile = wid * ngrp
    pltpu.sync_copy(col_hbm.at[pl.ds(g0_tile, ngrp)], cidx)
    if with_gather:
      pltpu.sync_copy(row_hbm.at[pl.ds(g0_tile, ngrp)], ridx)

    def zero_body(i, carry):
      stage[i, :] = jnp.zeros((LANES,), jnp.float32)
      return carry
    lax.fori_loop(0, zpt, zero_body, 0)
    pltpu.sync_copy(stage, agg.at[pl.ds(s * zpt, zpt)])

    if not with_gather:
      def ones_body(i, carry):
        msg[0, i, :] = jnp.ones((LANES,), jnp.float32)
        msg[1, i, :] = jnp.ones((LANES,), jnp.float32)
        return carry
      lax.fori_loop(0, CH, ones_body, 0)

    plsc.subcore_barrier()

    def chunk_body(t, carry):
      if with_gather:
        gathers = [
            pltpu.async_copy(zs_hbm.at[ridx.at[(2 * t + b) * KG + j]],
                             msg.at[b, pl.ds(j * GRP, GRP)], sem_g)
            for b in range(2) for j in range(KG)
        ]
      scatters = []
      for i, (b, j) in enumerate([(b, j) for b in range(2)
                                  for j in range(KG)]):
        if with_gather:
          gathers[i].wait()
        scatters.append(
            pltpu.async_copy(msg.at[b, pl.ds(j * GRP, GRP)],
                             agg.at[cidx.at[(2 * t + b) * KG + j]],
                             sem_s, add=True))
      for sc_ in scatters:
        sc_.wait()
      return carry
    lax.fori_loop(0, nch // 2, chunk_body, 0)

    plsc.subcore_barrier()
    pltpu.sync_copy(agg.at[pl.ds(s * zpt, zpt)], stage)
    pltpu.sync_copy(stage, out_hbm.at[c, pl.ds(s * zpt, zpt)])

  return sc_round


def _gelu(v):
  return 0.5 * v * (1.0 + lax.erf(v * (2.0 ** -0.5)))


def _ln(h, g, b):
  mu = jnp.mean(h, axis=-1, keepdims=True)
  d = h - mu
  var = jnp.mean(d * d, axis=-1, keepdims=True)
  return d * lax.rsqrt(var + 1e-5) * g + b


def _tc_pre_body(x_ref, w1_ref, b1_ref, g1_ref, bt1_ref, s0_ref,
                 zs_ref, h_ref, dinv_ref, ideg_ref):
  h = jnp.dot(x_ref[...], w1_ref[...], preferred_element_type=jnp.float32)
  h = _gelu(h + b1_ref[...])
  h = _ln(h, g1_ref[...], bt1_ref[...])
  deg = s0_ref[0] + s0_ref[1] + 1.0
  dinv = lax.rsqrt(deg)
  ideg = 1.0 / deg
  h_ref[...] = h
  zs_ref[...] = h * dinv
  dinv_ref[...] = dinv
  ideg_ref[...] = ideg


def _tc_mid_body(s1_ref, h_ref, dinv_ref, ideg_ref, zs1_ref, slf1_ref):
  h = h_ref[...]
  dinv = dinv_ref[...]
  ideg = ideg_ref[...]
  z1 = 0.5 * (dinv * (s1_ref[0] + s1_ref[1]) + h * ideg) + 0.5 * h
  zs1_ref[...] = z1 * dinv
  slf1_ref[...] = z1 * ideg


def _tc_post_body(s2_ref, h_ref, dinv_ref, slf1_ref, g2_ref, bt2_ref,
                  w2_ref, b2_ref, out_ref):
  h = h_ref[...]
  z2 = 0.5 * (dinv_ref[...] * (s2_ref[0] + s2_ref[1]) + slf1_ref[...]) + 0.5 * h
  t = _ln(_gelu(z2), g2_ref[...], bt2_ref[...])
  out_ref[...] = jnp.dot(t, w2_ref[...],
                         preferred_element_type=jnp.float32) + b2_ref[...]


def kernel(x, edge_index, W1, b1, g1, bt1, g2, bt2, W2, b2):
  n, din = x.shape
  hid = W1.shape[1]
  dout = W2.shape[1]
  assert hid == LANES
  e = edge_index.shape[1]

  span = NC * NS * CH
  e_pad = -(-e // span) * span
  pad = e_pad - e
  row = edge_index[0].astype(jnp.int32)
  col = edge_index[1].astype(jnp.int32)
  ar = jnp.arange(pad, dtype=jnp.int32)
  rowp = jnp.concatenate([row, ar % n]).reshape(e_pad // GRP, GRP)
  colp = jnp.concatenate([col, n + (ar % PADROWS)]).reshape(e_pad // GRP, GRP)
  n_agg = -(-(n + PADROWS) // (NS * 8)) * (NS * 8)
  ones = jnp.ones((n, LANES), jnp.float32)

  sc_deg = _make_sc_round(n_agg, e_pad, with_gather=False)
  sc_prop = _make_sc_round(n_agg, e_pad, with_gather=True)

  s0 = sc_deg(ones, rowp, colp)

  R = 2000
  assert n % R == 0
  grid = (n // R,)
  row_spec = pl.BlockSpec((R, LANES), lambda i: (i, 0))
  part_spec = pl.BlockSpec((NC, R, LANES), lambda i: (0, i, 0))
  vec16 = pl.BlockSpec((1, LANES), lambda i: (0, 0))
  st = jax.ShapeDtypeStruct((n, LANES), jnp.float32)

  zs0, h, dinv, ideg = pl.pallas_call(
      _tc_pre_body,
      grid=grid,
      in_specs=[
          pl.BlockSpec((R, din), lambda i: (i, 0)),
          pl.BlockSpec((din, LANES), lambda i: (0, 0)),
          vec16, vec16, vec16,
          part_spec,
      ],
      out_specs=[row_spec, row_spec, row_spec, row_spec],
      out_shape=[st, st, st, st],
  )(x, W1, b1.reshape(1, -1), g1.reshape(1, -1), bt1.reshape(1, -1), s0)

  s1 = sc_prop(zs0, rowp, colp)
  zs1, slf1 = pl.pallas_call(
      _tc_mid_body,
      grid=grid,
      in_specs=[part_spec, row_spec, row_spec, row_spec],
      out_specs=[row_spec, row_spec],
      out_shape=[st, st],
  )(s1, h, dinv, ideg)

  s2 = sc_prop(zs1, rowp, colp)
  out = pl.pallas_call(
      _tc_post_body,
      grid=grid,
      in_specs=[
          part_spec, row_spec, row_spec, row_spec,
          vec16, vec16,
          pl.BlockSpec((LANES, dout), lambda i: (0, 0)),
          pl.BlockSpec((1, dout), lambda i: (0, 0)),
      ],
      out_specs=pl.BlockSpec((R, dout), lambda i: (i, 0)),
      out_shape=jax.ShapeDtypeStruct((n, dout), jnp.float32),
  )(s2, h, dinv, slf1, g2.reshape(1, -1), bt2.reshape(1, -1),
    W2, b2.reshape(1, -1))
  return out


# speedup vs baseline: 1.5284x; 1.1165x over previous
import functools

import jax
import jax.numpy as jnp
from jax import lax
from jax.experimental import pallas as pl
from jax.experimental.pallas import tpu as pltpu
from jax.experimental.pallas import tpu_sc as plsc

NC = 2
NS = 16
LANES = 16
GRP = 128
CH = 1024
KG = CH // GRP
PADROWS = 64


def _make_sc_deg(n_agg, e_pad):
  nw = NC * NS
  ept = e_pad // nw
  nch = ept // CH
  assert ept % CH == 0 and n_agg % (NS * 8) == 0
  zpt = n_agg // NS
  ngrp = ept // GRP
  mesh = plsc.VectorSubcoreMesh(core_axis_name="c", subcore_axis_name="s")

  @functools.partial(
      pl.kernel,
      out_type=jax.ShapeDtypeStruct((NC, n_agg), jnp.float32),
      mesh=mesh,
      compiler_params=pltpu.CompilerParams(use_tc_tiling_on_sc=False),
      scratch_types=[
          pltpu.VMEM((ngrp, GRP), jnp.int32),
          pltpu.VMEM((GRP,), jnp.float32),
          pltpu.VMEM((n_agg // NS,), jnp.float32),
          pltpu.VMEM_SHARED((n_agg,), jnp.float32),
          pltpu.SemaphoreType.DMA,
      ],
  )
  def sc_deg(col_hbm, out_hbm, cidx, ones, stage, agg, sem_s):
    c = lax.axis_index("c")
    s = lax.axis_index("s")
    wid = s * NC + c

    pltpu.sync_copy(col_hbm.at[pl.ds(wid * ngrp, ngrp)], cidx)

    def zero_body(i, carry):
      stage[pl.ds(i * LANES, LANES)] = jnp.zeros((LANES,), jnp.float32)
      return carry
    lax.fori_loop(0, zpt // LANES, zero_body, 0)
    pltpu.sync_copy(stage, agg.at[pl.ds(s * zpt, zpt)])

    def ones_body(i, carry):
      ones[pl.ds(i * LANES, LANES)] = jnp.ones((LANES,), jnp.float32)
      return carry
    lax.fori_loop(0, GRP // LANES, ones_body, 0)
    plsc.subcore_barrier()

    prev = []
    for t in range(nch):
      cur = [
          pltpu.async_copy(ones, agg.at[cidx.at[t * KG + j]],
                           sem_s, add=True)
          for j in range(KG)
      ]
      for d in prev:
        d.wait()
      prev = cur
    for d in prev:
      d.wait()

    plsc.subcore_barrier()
    pltpu.sync_copy(agg.at[pl.ds(s * zpt, zpt)], stage)
    pltpu.sync_copy(stage, out_hbm.at[c, pl.ds(s * zpt, zpt)])

  return sc_deg


def _make_sc_round(n_agg, e_pad, with_gather):
  nw = NC * NS
  ept = e_pad // nw
  nch = ept // CH
  assert ept % (2 * CH) == 0 and n_agg % (NS * 8) == 0
  zpt = n_agg // NS
  mesh = plsc.VectorSubcoreMesh(core_axis_name="c", subcore_axis_name="s")

  ngrp = ept // GRP

  @functools.partial(
      pl.kernel,
      out_type=jax.ShapeDtypeStruct((NC, n_agg, LANES), jnp.float32),
      mesh=mesh,
      compiler_params=pltpu.CompilerParams(use_tc_tiling_on_sc=False),
      scratch_types=[
          pltpu.VMEM((ngrp, GRP), jnp.int32),
          pltpu.VMEM((ngrp, GRP), jnp.int32),
          pltpu.VMEM((2, CH, LANES), jnp.float32),
          pltpu.VMEM((n_agg // NS, LANES), jnp.float32),
          pltpu.VMEM_SHARED((n_agg, LANES), jnp.float32),
          pltpu.SemaphoreType.DMA,
          pltpu.SemaphoreType.DMA,
      ],
  )
  def sc_round(zs_hbm, row_hbm, col_hbm, out_hbm,
               ridx, cidx, msg, stage, agg, sem_g, sem_s):
    c = lax.axis_index("c")
    s = lax.axis_index("s")
    wid = s * NC + c

    g0_tile = wid * ngrp
    pltpu.sync_copy(col_hbm.at[pl.ds(g0_tile, ngrp)], cidx)
    if with_gather:
      pltpu.sync_copy(row_hbm.at[pl.ds(g0_tile, ngrp)], ridx)

    def zero_body(i, carry):
      stage[i, :] = jnp.zeros((LANES,), jnp.float32)
      return carry
    lax.fori_loop(0, zpt, zero_body, 0)
    pltpu.sync_copy(stage, agg.at[pl.ds(s * zpt, zpt)])

    if not with_gather:
      def ones_body(i, carry):
        msg[0, i, :] = jnp.ones((LANES,), jnp.float32)
        msg[1, i, :] = jnp.ones((LANES,), jnp.float32)
        return carry
      lax.fori_loop(0, CH, ones_body, 0)

    plsc.subcore_barrier()

    def chunk_body(t, carry):
      if with_gather:
        gathers = [
            pltpu.async_copy(zs_hbm.at[ridx.at[(2 * t + b) * KG + j]],
                             msg.at[b, pl.ds(j * GRP, GRP)], sem_g)
            for b in range(2) for j in range(KG)
        ]
      scatters = []
      for i, (b, j) in enumerate([(b, j) for b in range(2)
                                  for j in range(KG)]):
        if with_gather:
          gathers[i].wait()
        scatters.append(
            pltpu.async_copy(msg.at[b, pl.ds(j * GRP, GRP)],
                             agg.at[cidx.at[(2 * t + b) * KG + j]],
                             sem_s, add=True))
      for sc_ in scatters:
        sc_.wait()
      return carry
    lax.fori_loop(0, nch // 2, chunk_body, 0)

    plsc.subcore_barrier()
    pltpu.sync_copy(agg.at[pl.ds(s * zpt, zpt)], stage)
    pltpu.sync_copy(stage, out_hbm.at[c, pl.ds(s * zpt, zpt)])

  return sc_round


def _gelu(v):
  return 0.5 * v * (1.0 + lax.erf(v * (2.0 ** -0.5)))


def _ln(h, g, b):
  mu = jnp.mean(h, axis=-1, keepdims=True)
  d = h - mu
  var = jnp.mean(d * d, axis=-1, keepdims=True)
  return d * lax.rsqrt(var + 1e-5) * g + b


def _tc_pre_body(x_ref, w1_ref, b1_ref, g1_ref, bt1_ref, s0_ref,
                 zs_ref, h_ref, dinv_ref, ideg_ref):
  h = jnp.dot(x_ref[...], w1_ref[...], preferred_element_type=jnp.float32)
  h = _gelu(h + b1_ref[...])
  h = _ln(h, g1_ref[...], bt1_ref[...])
  deg = jnp.broadcast_to((s0_ref[0] + s0_ref[1] + 1.0)[:, None], h.shape)
  dinv = lax.rsqrt(deg)
  ideg = 1.0 / deg
  h_ref[...] = h
  zs_ref[...] = h * dinv
  dinv_ref[...] = dinv
  ideg_ref[...] = ideg


def _tc_mid_body(s1_ref, h_ref, dinv_ref, ideg_ref, zs1_ref, slf1_ref):
  h = h_ref[...]
  dinv = dinv_ref[...]
  ideg = ideg_ref[...]
  z1 = 0.5 * (dinv * (s1_ref[0] + s1_ref[1]) + h * ideg) + 0.5 * h
  zs1_ref[...] = z1 * dinv
  slf1_ref[...] = z1 * ideg


def _tc_post_body(s2_ref, h_ref, dinv_ref, slf1_ref, g2_ref, bt2_ref,
                  w2_ref, b2_ref, out_ref):
  h = h_ref[...]
  z2 = 0.5 * (dinv_ref[...] * (s2_ref[0] + s2_ref[1]) + slf1_ref[...]) + 0.5 * h
  t = _ln(_gelu(z2), g2_ref[...], bt2_ref[...])
  out_ref[...] = jnp.dot(t, w2_ref[...],
                         preferred_element_type=jnp.float32) + b2_ref[...]


def kernel(x, edge_index, W1, b1, g1, bt1, g2, bt2, W2, b2):
  n, din = x.shape
  hid = W1.shape[1]
  dout = W2.shape[1]
  assert hid == LANES
  e = edge_index.shape[1]

  span = NC * NS * CH
  e_pad = -(-e // span) * span
  pad = e_pad - e
  row = edge_index[0].astype(jnp.int32)
  col = edge_index[1].astype(jnp.int32)
  ar = jnp.arange(pad, dtype=jnp.int32)
  rowp = jnp.concatenate([row, ar % n]).reshape(e_pad // GRP, GRP)
  colp = jnp.concatenate([col, n + (ar % PADROWS)]).reshape(e_pad // GRP, GRP)
  n_agg = -(-(n + PADROWS) // (NS * 8)) * (NS * 8)

  sc_prop = _make_sc_round(n_agg, e_pad, with_gather=True)

  s0 = _make_sc_deg(n_agg, e_pad)(colp)

  R = 2048
  grid = (-(-n_agg // R),)
  row_spec = pl.BlockSpec((R, LANES), lambda i: (i, 0))
  part_spec = pl.BlockSpec((NC, R, LANES), lambda i: (0, i, 0))
  vec16 = pl.BlockSpec((1, LANES), lambda i: (0, 0))
  st = jax.ShapeDtypeStruct((n, LANES), jnp.float32)

  zs0, h, dinv, ideg = pl.pallas_call(
      _tc_pre_body,
      grid=grid,
      in_specs=[
          pl.BlockSpec((R, din), lambda i: (i, 0)),
          pl.BlockSpec((din, LANES), lambda i: (0, 0)),
          vec16, vec16, vec16,
          pl.BlockSpec((NC, R), lambda i: (0, i)),
      ],
      out_specs=[row_spec, row_spec, row_spec, row_spec],
      out_shape=[st, st, st, st],
  )(x, W1, b1.reshape(1, -1), g1.reshape(1, -1), bt1.reshape(1, -1), s0)

  s1 = sc_prop(zs0, rowp, colp)
  zs1, slf1 = pl.pallas_call(
      _tc_mid_body,
      grid=grid,
      in_specs=[part_spec, row_spec, row_spec, row_spec],
      out_specs=[row_spec, row_spec],
      out_shape=[st, st],
  )(s1, h, dinv, ideg)

  s2 = sc_prop(zs1, rowp, colp)
  out = pl.pallas_call(
      _tc_post_body,
      grid=grid,
      in_specs=[
          part_spec, row_spec, row_spec, row_spec,
          vec16, vec16,
          pl.BlockSpec((LANES, dout), lambda i: (0, 0)),
          pl.BlockSpec((1, dout), lambda i: (0, 0)),
      ],
      out_specs=pl.BlockSpec((R, dout), lambda i: (i, 0)),
      out_shape=jax.ShapeDtypeStruct((n, dout), jnp.float32),
  )(s2, h, dinv, slf1, g2.reshape(1, -1), bt2.reshape(1, -1),
    W2, b2.reshape(1, -1))
  return out
